# Initial kernel scaffold; baseline (speedup 1.0000x reference)
import jax
import jax.numpy as jnp
from jax.experimental import pallas as pl


def kernel(inputs, laplacian, w1_in, w1_h, b1, w2_in, w2_h, b2):
    raise NotImplementedError("write your pallas kernel here")


# speedup vs baseline: 1.5932x; 1.5932x over previous
import jax
import jax.numpy as jnp
from jax import lax
from jax.experimental import pallas as pl
from jax.experimental.pallas import tpu as pltpu


def _build_kernel(batch, seq_len, n, hd, k_blk):
    nh = n // 2

    def _body(x_ref, a1_ref, a2_ref, lcpt_ref, lppt_ref, w1t_ref, w2t_ref,
              w1bc_ref, b1bc_ref, w2bc_ref, b2bc_ref, out_ref,
              lx_ref, lxp_ref):
        xb = x_ref[...]
        lx_ref[...] = jnp.dot(xb, a1_ref[...],
                              preferred_element_type=jnp.float32)
        lxp_ref[...] = jnp.dot(xb, a2_ref[...],
                               preferred_element_type=jnp.float32)

        lcpt = lcpt_ref[...]
        lppt = lppt_ref[...]
        w1t = w1t_ref[...]
        w2t = w2t_ref[...]
        w1bc = w1bc_ref[...]
        b1bc = b1bc_ref[...]
        w2bc = w2bc_ref[...]
        b2bc = b2bc_ref[...]

        def step(t, hs):
            new_hs = []
            for k in range(k_blk):
                h = hs[k]
                lht = jnp.dot(h, lcpt, preferred_element_type=jnp.float32)
                row = k * seq_len + t
                lxr = lx_ref[pl.ds(row, 1), :]
                xb1t = w1bc * lxr + b1bc
                g = jax.nn.sigmoid(
                    xb1t + jnp.dot(w1t, lht,
                                   preferred_element_type=jnp.float32))
                r = jnp.concatenate([g[:hd, :nh], g[hd:, :nh]], axis=1)
                u = jnp.concatenate([g[:hd, nh:], g[hd:, nh:]], axis=1)
                agg = jnp.dot(r * h, lppt,
                              preferred_element_type=jnp.float32)
                lxpr = lxp_ref[pl.ds(row, 1), :]
                xb2t = w2bc * lxpr + b2bc
                c = jnp.tanh(
                    xb2t + jnp.dot(w2t, agg,
                                   preferred_element_type=jnp.float32))
                new_hs.append(u * h + (1.0 - u) * c)
            return tuple(new_hs)

        hs0 = tuple(jnp.zeros((hd, n), jnp.float32) for _ in range(k_blk))
        hs = lax.fori_loop(0, seq_len, step, hs0)
        for k in range(k_blk):
            out_ref[k, :, :] = hs[k]

    grid = (batch // k_blk,)
    kt = k_blk * seq_len
    return pl.pallas_call(
        _body,
        grid=grid,
        in_specs=[
            pl.BlockSpec((kt, n), lambda b: (b, 0)),
            pl.BlockSpec((n, n), lambda b: (0, 0)),
            pl.BlockSpec((n, n), lambda b: (0, 0)),
            pl.BlockSpec((n, n), lambda b: (0, 0)),
            pl.BlockSpec((n, n), lambda b: (0, 0)),
            pl.BlockSpec((2 * hd, hd), lambda b: (0, 0)),
            pl.BlockSpec((hd, hd), lambda b: (0, 0)),
            pl.BlockSpec((2 * hd, n), lambda b: (0, 0)),
            pl.BlockSpec((2 * hd, n), lambda b: (0, 0)),
            pl.BlockSpec((hd, n), lambda b: (0, 0)),
            pl.BlockSpec((hd, n), lambda b: (0, 0)),
        ],
        out_specs=pl.BlockSpec((k_blk, hd, n), lambda b: (b, 0, 0)),
        out_shape=jax.ShapeDtypeStruct((batch, hd, n), jnp.float32),
        scratch_shapes=[
            pltpu.VMEM((kt, n), jnp.float32),
            pltpu.VMEM((kt, n), jnp.float32),
        ],
        compiler_params=pltpu.CompilerParams(
            dimension_semantics=("parallel",)),
    )


def kernel(inputs, laplacian, w1_in, w1_h, b1, w2_in, w2_h, b2):
    inputs = inputs.astype(jnp.float32)
    b, seq_len, n = inputs.shape
    hd = w1_h.shape[0]
    k_blk = 8 if b % 8 == 0 else (4 if b % 4 == 0 else (2 if b % 2 == 0 else 1))

    lap = laplacian.astype(jnp.float32)
    perm = jnp.concatenate([jnp.arange(0, n, 2), jnp.arange(1, n, 2)])
    a1 = lap.T
    a2 = a1[:, perm]
    lcpt = a1[perm, :]
    lppt = a1[perm][:, perm]

    w1t = w1_h.astype(jnp.float32).T
    w2t = w2_h.astype(jnp.float32).T
    w1bc = jnp.broadcast_to(w1_in.reshape(-1)[:, None], (2 * hd, n))
    b1bc = jnp.broadcast_to(b1.reshape(-1)[:, None], (2 * hd, n))
    w2bc = jnp.broadcast_to(w2_in.reshape(-1)[:, None], (hd, n))
    b2bc = jnp.broadcast_to(b2.reshape(-1)[:, None], (hd, n))

    x2 = inputs.reshape(b * seq_len, n)
    fused = _build_kernel(b, seq_len, n, hd, k_blk)
    out_t = fused(x2, a1, a2, lcpt, lppt, w1t, w2t, w1bc, b1bc, w2bc, b2bc)

    inv = jnp.argsort(perm)
    return jnp.transpose(out_t, (0, 2, 1))[:, inv, :]


# speedup vs baseline: 4.6481x; 2.9174x over previous
import jax
import jax.numpy as jnp
from jax import lax
from jax.experimental import pallas as pl
from jax.experimental.pallas import tpu as pltpu

_KG = 4


def _build_kernel(batch, seq_len, n, hd, n_grp):
    nh = n // 2
    blk = _KG * n_grp

    def _body(x_ref, a1_ref, a2_ref, lcpt_ref, lppt_ref, w1bd_ref, w2bd_ref,
              w1bc_ref, b1bc_ref, w2bc_ref, b2bc_ref, out_ref,
              lx_ref, lxp_ref):
        xb = x_ref[...]
        lx_ref[...] = jnp.dot(xb, a1_ref[...],
                              preferred_element_type=jnp.float32)
        lxp_ref[...] = jnp.dot(xb, a2_ref[...],
                               preferred_element_type=jnp.float32)

        lcpt = lcpt_ref[...]
        lppt = lppt_ref[...]
        w1bd = w1bd_ref[...]
        w2bd = w2bd_ref[...]
        w1bc = w1bc_ref[...]
        b1bc = b1bc_ref[...]
        w2bc = w2bc_ref[...]
        b2bc = b2bc_ref[...]

        def step(t, hs):
            lhts = [jnp.dot(hs[g], lcpt, preferred_element_type=jnp.float32)
                    for g in range(n_grp)]
            gpre = [jnp.dot(w1bd, lhts[g], preferred_element_type=jnp.float32)
                    for g in range(n_grp)]
            rts, uts = [], []
            for g in range(n_grp):
                xb1 = jnp.concatenate(
                    [w1bc * lx_ref[pl.ds((_KG * g + k) * seq_len + t, 1), :]
                     + b1bc for k in range(_KG)], axis=0)
                gt = jax.nn.sigmoid(xb1 + gpre[g])
                rs, us = [], []
                for k in range(_KG):
                    gk = gt[2 * hd * k:2 * hd * (k + 1)]
                    rs.append(jnp.concatenate(
                        [gk[:hd, :nh], gk[hd:, :nh]], axis=1))
                    us.append(jnp.concatenate(
                        [gk[:hd, nh:], gk[hd:, nh:]], axis=1))
                rts.append(jnp.concatenate(rs, axis=0))
                uts.append(jnp.concatenate(us, axis=0))
            aggs = [jnp.dot(rts[g] * hs[g], lppt,
                            preferred_element_type=jnp.float32)
                    for g in range(n_grp)]
            cpre = [jnp.dot(w2bd, aggs[g], preferred_element_type=jnp.float32)
                    for g in range(n_grp)]
            new_hs = []
            for g in range(n_grp):
                xb2 = jnp.concatenate(
                    [w2bc * lxp_ref[pl.ds((_KG * g + k) * seq_len + t, 1), :]
                     + b2bc for k in range(_KG)], axis=0)
                ct = jnp.tanh(xb2 + cpre[g])
                u = uts[g]
                new_hs.append(u * hs[g] + (1.0 - u) * ct)
            return tuple(new_hs)

        hs0 = tuple(jnp.zeros((_KG * hd, n), jnp.float32)
                    for _ in range(n_grp))
        hs = lax.fori_loop(0, seq_len, step, hs0)
        for g in range(n_grp):
            for k in range(_KG):
                out_ref[_KG * g + k, :, :] = hs[g][hd * k:hd * (k + 1)]

    grid = (batch // blk,)
    kt = blk * seq_len
    return pl.pallas_call(
        _body,
        grid=grid,
        in_specs=[
            pl.BlockSpec((kt, n), lambda b: (b, 0)),
            pl.BlockSpec((n, n), lambda b: (0, 0)),
            pl.BlockSpec((n, n), lambda b: (0, 0)),
            pl.BlockSpec((n, n), lambda b: (0, 0)),
            pl.BlockSpec((n, n), lambda b: (0, 0)),
            pl.BlockSpec((_KG * 2 * hd, _KG * hd), lambda b: (0, 0)),
            pl.BlockSpec((_KG * hd, _KG * hd), lambda b: (0, 0)),
            pl.BlockSpec((2 * hd, n), lambda b: (0, 0)),
            pl.BlockSpec((2 * hd, n), lambda b: (0, 0)),
            pl.BlockSpec((hd, n), lambda b: (0, 0)),
            pl.BlockSpec((hd, n), lambda b: (0, 0)),
        ],
        out_specs=pl.BlockSpec((blk, hd, n), lambda b: (b, 0, 0)),
        out_shape=jax.ShapeDtypeStruct((batch, hd, n), jnp.float32),
        scratch_shapes=[
            pltpu.VMEM((kt, n), jnp.float32),
            pltpu.VMEM((kt, n), jnp.float32),
        ],
        compiler_params=pltpu.CompilerParams(
            dimension_semantics=("parallel",)),
    )


def kernel(inputs, laplacian, w1_in, w1_h, b1, w2_in, w2_h, b2):
    inputs = inputs.astype(jnp.float32)
    b, seq_len, n = inputs.shape
    hd = w1_h.shape[0]
    n_grp = 2 if b % (2 * _KG) == 0 else 1

    lap = laplacian.astype(jnp.float32)
    perm = jnp.concatenate([jnp.arange(0, n, 2), jnp.arange(1, n, 2)])
    a1 = lap.T
    a2 = a1[:, perm]
    lcpt = a1[perm, :]
    lppt = a1[perm][:, perm]

    w1t = w1_h.astype(jnp.float32).T
    w2t = w2_h.astype(jnp.float32).T
    w1bd = jax.scipy.linalg.block_diag(*([w1t] * _KG))
    w2bd = jax.scipy.linalg.block_diag(*([w2t] * _KG))
    w1bc = jnp.broadcast_to(w1_in.reshape(-1)[:, None], (2 * hd, n))
    b1bc = jnp.broadcast_to(b1.reshape(-1)[:, None], (2 * hd, n))
    w2bc = jnp.broadcast_to(w2_in.reshape(-1)[:, None], (hd, n))
    b2bc = jnp.broadcast_to(b2.reshape(-1)[:, None], (hd, n))

    x2 = inputs.reshape(b * seq_len, n)
    fused = _build_kernel(b, seq_len, n, hd, n_grp)
    out_t = fused(x2, a1, a2, lcpt, lppt, w1bd, w2bd,
                  w1bc, b1bc, w2bc, b2bc)

    inv = jnp.argsort(perm)
    return jnp.transpose(out_t, (0, 2, 1))[:, inv, :]
